# baseline (speedup 1.0000x reference)
import jax
import jax.numpy as jnp
from jax import lax
from jax.experimental import pallas as pl
from jax.experimental.pallas import tpu as pltpu
from jax.experimental.pallas import tpu_sc as plsc

NROWS = 100000
D = 512
NSEG = 512
L = 16
NCB = 8
B = 160
GPB = B // L
SHARD = 12800
NBLK7 = (NROWS - 7 * SHARD) // B
NBLKX = SHARD // B
SLAB = NSEG * 2 * 128


def _sc_body(x_hbm, ids_hbm, out_hbm, scr_hbm,
             xbuf0, xbuf1, ibuf0, ibuf1, sacc, macc, fbufA, fbufB, obuf,
             osum2d, omax2d,
             mbufs0, mbufs1, mbufs2, mbufs3, mbufs4, mbufs5, mbufs6, mbufs7,
             nflush,
             xsem0, xsem1, isem0, isem1, fsem0, fsem1, msem):
    c = lax.axis_index("c")
    s = lax.axis_index("s")
    cg = c * 2 + s // 8
    rs = s % 8
    mbufs = [mbufs0, mbufs1, mbufs2, mbufs3, mbufs4, mbufs5, mbufs6, mbufs7]

    ccol = pl.multiple_of(cg * 128, 128)
    rbase = pl.multiple_of(rs * SHARD, 8)
    nblk = jnp.where(rs == 7, NBLK7, NBLKX)
    sbase = pl.multiple_of((cg * 8 + rs) * SLAB, 8)

    zeros = jnp.zeros((L,), jnp.float32)
    ninf = jnp.full((L,), -jnp.inf, jnp.float32)

    for r in range(32):
        for k in range(NCB):
            obuf[pl.ds(r * 256 + k * L, L)] = zeros
            obuf[pl.ds(r * 256 + 128 + k * L, L)] = ninf
    for chunk in range(16):
        pltpu.sync_copy(
            obuf,
            scr_hbm.at[pl.ds(pl.multiple_of(sbase + chunk * 8192, 8), 8192)])
    nflush[0] = 0

    def xstart(g, xb, ib, xsem, isem):
        r0 = pl.multiple_of(rbase + g * B, 8)
        pltpu.async_copy(x_hbm.at[pl.ds(r0, B), pl.ds(ccol, 128)], xb, xsem)
        pltpu.async_copy(ids_hbm.at[pl.ds(r0, B)], ib, isem)

    def xwait(g, xb, ib, xsem, isem):
        r0 = pl.multiple_of(rbase + g * B, 8)
        pltpu.make_async_copy(
            x_hbm.at[pl.ds(r0, B), pl.ds(ccol, 128)], xb, xsem).wait()
        pltpu.make_async_copy(ids_hbm.at[pl.ds(r0, B)], ib, isem).wait()

    def _tree(op, xs):
        while len(xs) > 1:
            xs = [op(xs[i], xs[i + 1]) for i in range(0, len(xs), 2)]
        return xs[0]

    def flush(seg, svals, mvals):
        n = nflush[0]
        dst = scr_hbm.at[pl.ds(pl.multiple_of(sbase + seg * 256, 8), 256)]
        for slot, fb, fsem in ((0, fbufA, fsem0), (1, fbufB, fsem1)):
            @pl.when(lax.rem(n, 2) == slot)
            def _(fb=fb, fsem=fsem):
                @pl.when(n >= 2)
                def _():
                    pltpu.make_async_copy(fb, dst, fsem).wait()

                for k in range(NCB):
                    fb[pl.ds(k * L, L)] = svals[k]
                    fb[pl.ds(128 + k * L, L)] = mvals[k]
                pltpu.async_copy(fb, dst, fsem)
        nflush[0] = n + 1

    def group(xb, ib, base, prev, sa, ma):
        idvec = ib[pl.ds(base, L)]
        same = jnp.logical_and(idvec[0] == idvec[L - 1], idvec[0] == prev)

        @pl.when(jnp.logical_not(same))
        def slow():
            p = prev
            sa_s = list(sa)
            ma_s = list(ma)
            for j in range(L):
                sid = idvec[j]
                new = sid != p

                @pl.when(jnp.logical_and(new, p >= 0))
                def _(sa_s=tuple(sa_s), ma_s=tuple(ma_s), p=p):
                    flush(p, sa_s, ma_s)

                vs = [xb[base + j, pl.ds(k * L, L)] for k in range(NCB)]
                for k in range(NCB):
                    sa_s[k] = jnp.where(new, vs[k], sa_s[k] + vs[k])
                    ma_s[k] = jnp.where(new, vs[k], jnp.maximum(ma_s[k], vs[k]))
                p = sid
            for k in range(NCB):
                sacc[pl.ds(k * L, L)] = sa_s[k]
                macc[pl.ds(k * L, L)] = ma_s[k]

        sa_n, ma_n = [], []
        for k in range(NCB):
            vs = [xb[base + j, pl.ds(k * L, L)] for j in range(L)]
            S = _tree(lambda a, b: a + b, vs)
            M = _tree(jnp.maximum, vs)
            sa_n.append(jnp.where(same, sa[k] + S, sacc[pl.ds(k * L, L)]))
            ma_n.append(jnp.where(same, jnp.maximum(ma[k], M),
                                  macc[pl.ds(k * L, L)]))
        return idvec[L - 1], sa_n, ma_n

    def process(xb, ib, carry):
        def gpair(k, carry):
            prev = carry[0]
            sa = list(carry[1:1 + NCB])
            ma = list(carry[1 + NCB:])
            prev, sa, ma = group(xb, ib, k * (2 * L), prev, sa, ma)
            prev, sa, ma = group(xb, ib, k * (2 * L) + L, prev, sa, ma)
            return (prev, *sa, *ma)
        return lax.fori_loop(0, GPB // 2, gpair, carry)

    xstart(0, xbuf0, ibuf0, xsem0, isem0)
    carry = (jnp.int32(-1), *([zeros] * NCB), *([ninf] * NCB))

    def pair_body(p, carry):
        g0 = 2 * p
        g1 = g0 + 1

        @pl.when(g1 < nblk)
        def _():
            xstart(g1, xbuf1, ibuf1, xsem1, isem1)

        xwait(g0, xbuf0, ibuf0, xsem0, isem0)
        carry = process(xbuf0, ibuf0, carry)

        @pl.when(g0 + 2 < nblk)
        def _():
            xstart(g0 + 2, xbuf0, ibuf0, xsem0, isem0)

        @pl.when(g1 < nblk)
        def _2():
            xwait(g1, xbuf1, ibuf1, xsem1, isem1)

        @pl.when(g1 >= nblk)
        def _3():
            pid = jnp.full((L,), carry[0], jnp.int32)
            for i in range(GPB):
                ibuf1[pl.ds(i * L, L)] = pid

        carry2 = process(xbuf1, ibuf1, carry)
        carry = jax.tree.map(
            lambda a, b: jnp.where(g1 < nblk, b, a), carry, carry2)
        return carry

    npair = (nblk + 1) // 2
    carry = lax.fori_loop(0, npair, pair_body, carry)
    prev = carry[0]
    sa = list(carry[1:1 + NCB])
    ma = list(carry[1 + NCB:])

    flush(prev, sa, ma)
    n = nflush[0]
    for slot, fb, fsem in ((0, fbufA, fsem0), (1, fbufB, fsem1)):
        @pl.when(jnp.logical_or(
            n >= 2, jnp.logical_and(n >= 1, lax.rem(n - 1, 2) == slot)))
        def _(fb=fb, fsem=fsem):
            pltpu.make_async_copy(
                fb, scr_hbm.at[pl.ds(pl.multiple_of(sbase, 8), 256)],
                fsem).wait()

    plsc.subcore_barrier()

    seg0 = rs * 64
    for half in range(2):
        segh = pl.multiple_of(seg0 + half * 32, 8)
        for sh in range(8):
            src = scr_hbm.at[pl.ds(
                pl.multiple_of((cg * 8 + sh) * SLAB + segh * 256, 8), 8192)]
            pltpu.async_copy(src, mbufs[sh], msem)
        for sh in range(8):
            src = scr_hbm.at[pl.ds(
                pl.multiple_of((cg * 8 + sh) * SLAB + segh * 256, 8), 8192)]
            pltpu.make_async_copy(src, mbufs[sh], msem).wait()

        def mrow(r, carry):
            for k in range(NCB):
                vals = [mbufs[sh][pl.ds(r * 256 + k * L, L)]
                        for sh in range(8)]
                osum2d[r, pl.ds(k * L, L)] = _tree(lambda a, b: a + b, vals)
                vals = [mbufs[sh][pl.ds(r * 256 + 128 + k * L, L)]
                        for sh in range(8)]
                omax2d[r, pl.ds(k * L, L)] = _tree(jnp.maximum, vals)
            return carry

        lax.fori_loop(0, 32, mrow, 0)
        pltpu.sync_copy(osum2d,
                        out_hbm.at[pl.ds(segh, 32), pl.ds(ccol, 128)])
        pltpu.sync_copy(
            omax2d,
            out_hbm.at[pl.ds(segh, 32),
                       pl.ds(pl.multiple_of(D + cg * 128, 128), 128)])


@jax.jit
def _readout(x, ids):
    mesh = plsc.VectorSubcoreMesh(core_axis_name="c", subcore_axis_name="s")
    fn = pl.kernel(
        _sc_body,
        out_type=(jax.ShapeDtypeStruct((NSEG, 2 * D), jnp.float32),
                  jax.ShapeDtypeStruct((32 * SLAB,), jnp.float32)),
        mesh=mesh,
        scratch_types=[
            pltpu.VMEM((B, 128), jnp.float32),
            pltpu.VMEM((B, 128), jnp.float32),
            pltpu.VMEM((B,), jnp.int32),
            pltpu.VMEM((B,), jnp.int32),
            pltpu.VMEM((128,), jnp.float32),
            pltpu.VMEM((128,), jnp.float32),
            pltpu.VMEM((256,), jnp.float32),
            pltpu.VMEM((256,), jnp.float32),
            pltpu.VMEM((8192,), jnp.float32),
            pltpu.VMEM((32, 128), jnp.float32),
            pltpu.VMEM((32, 128), jnp.float32),
            pltpu.VMEM((8192,), jnp.float32),
            pltpu.VMEM((8192,), jnp.float32),
            pltpu.VMEM((8192,), jnp.float32),
            pltpu.VMEM((8192,), jnp.float32),
            pltpu.VMEM((8192,), jnp.float32),
            pltpu.VMEM((8192,), jnp.float32),
            pltpu.VMEM((8192,), jnp.float32),
            pltpu.VMEM((8192,), jnp.float32),
            pltpu.SMEM((1,), jnp.int32),
            pltpu.SemaphoreType.DMA,
            pltpu.SemaphoreType.DMA,
            pltpu.SemaphoreType.DMA,
            pltpu.SemaphoreType.DMA,
            pltpu.SemaphoreType.DMA,
            pltpu.SemaphoreType.DMA,
            pltpu.SemaphoreType.DMA,
        ],
    )
    out, _ = fn(x, ids)
    return out


def kernel(x, batch):
    return _readout(x, batch.astype(jnp.int32))


# speedup vs baseline: 1.2807x; 1.2807x over previous
import jax
import jax.numpy as jnp
from jax import lax
from jax.experimental import pallas as pl
from jax.experimental.pallas import tpu as pltpu
from jax.experimental.pallas import tpu_sc as plsc

NROWS = 100000
D = 512
NSEG = 512
L = 16
NCB = 8
BI = 20
B = BI * 8
GPB = B // L
SHARDI = 1600
NBLK7 = (NROWS // 8 - 7 * SHARDI) // BI
NBLKX = SHARDI // BI
SLAB = NSEG * 2 * 128


def _sc_body(x_hbm, ids_hbm, out_hbm, scr_hbm,
             xbuf0, xbuf1, ibuf0, ibuf1, sacc, macc, fbufA, fbufB, obuf,
             osum2d, omax2d,
             mbufs0, mbufs1, mbufs2, mbufs3, mbufs4, mbufs5, mbufs6, mbufs7,
             nflush, pseg,
             xsem0, xsem1, isem0, isem1, fsem0, fsem1, msem):
    c = lax.axis_index("c")
    s = lax.axis_index("s")
    cg = c * 2 + s // 8
    rs = s % 8
    mbufs = [mbufs0, mbufs1, mbufs2, mbufs3, mbufs4, mbufs5, mbufs6, mbufs7]

    ccol = pl.multiple_of(cg * 128, 128)
    ibase = rs * SHARDI
    nblk = jnp.where(rs == 7, NBLK7, NBLKX)
    sbase = pl.multiple_of((cg * 8 + rs) * SLAB, 8)

    zeros = jnp.zeros((L,), jnp.float32)
    ninf = jnp.full((L,), -jnp.inf, jnp.float32)

    for r in range(32):
        for k in range(NCB):
            obuf[pl.ds(r * 256 + k * L, L)] = zeros
            obuf[pl.ds(r * 256 + 128 + k * L, L)] = ninf
    for chunk in range(16):
        pltpu.sync_copy(
            obuf,
            scr_hbm.at[pl.ds(pl.multiple_of(sbase + chunk * 8192, 8), 8192)])
    nflush[0] = 0
    pseg[0] = -1
    for k in range(NCB):
        sacc[pl.ds(k * L, L)] = zeros
        macc[pl.ds(k * L, L)] = ninf

    def xstart(g, xb, ib, xsem, isem):
        i0 = ibase + g * BI
        pltpu.async_copy(x_hbm.at[pl.ds(i0, BI), :, pl.ds(ccol, 128)],
                         xb, xsem)
        pltpu.async_copy(
            ids_hbm.at[pl.ds(pl.multiple_of(i0 * 8, 8), B)], ib, isem)

    def xwait(g, xb, ib, xsem, isem):
        i0 = ibase + g * BI
        pltpu.make_async_copy(
            x_hbm.at[pl.ds(i0, BI), :, pl.ds(ccol, 128)], xb, xsem).wait()
        pltpu.make_async_copy(
            ids_hbm.at[pl.ds(pl.multiple_of(i0 * 8, 8), B)], ib, isem).wait()

    def _tree(op, xs):
        while len(xs) > 1:
            xs = [op(xs[i], xs[i + 1]) for i in range(0, len(xs), 2)]
        return xs[0]

    def flush(seg, svals, mvals):
        n = nflush[0]
        dst = scr_hbm.at[pl.ds(pl.multiple_of(sbase + seg * 256, 8), 256)]
        for slot, fb, fsem in ((0, fbufA, fsem0), (1, fbufB, fsem1)):
            @pl.when(lax.rem(n, 2) == slot)
            def _(fb=fb, fsem=fsem):
                @pl.when(n >= 2)
                def _():
                    pltpu.make_async_copy(fb, dst, fsem).wait()

                for k in range(NCB):
                    fb[pl.ds(k * L, L)] = svals[k]
                    fb[pl.ds(128 + k * L, L)] = mvals[k]
                pltpu.async_copy(fb, dst, fsem)
        nflush[0] = n + 1

    def group(xb, ib, gi, i2):
        idvec = ib[pl.ds(gi * L, L)]
        prev = pseg[0]
        same = jnp.logical_and(idvec[0] == idvec[L - 1], idvec[0] == prev)

        @pl.when(same)
        def fast():
            for k in range(NCB):
                vs = [xb[i2 + a, r, pl.ds(k * L, L)]
                      for a in range(2) for r in range(8)]
                plsc.addupdate(sacc.at[pl.ds(k * L, L)],
                               _tree(lambda x, y: x + y, vs))
                macc[pl.ds(k * L, L)] = jnp.maximum(
                    macc[pl.ds(k * L, L)], _tree(jnp.maximum, vs))

        @pl.when(jnp.logical_not(same))
        def slow():
            p = prev
            sa_s = [sacc[pl.ds(k * L, L)] for k in range(NCB)]
            ma_s = [macc[pl.ds(k * L, L)] for k in range(NCB)]
            for j in range(L):
                a, r = j // 8, j % 8
                sid = idvec[j]
                new = sid != p

                @pl.when(jnp.logical_and(new, p >= 0))
                def _(sa_s=tuple(sa_s), ma_s=tuple(ma_s), p=p):
                    flush(p, sa_s, ma_s)

                vs = [xb[i2 + a, r, pl.ds(k * L, L)] for k in range(NCB)]
                for k in range(NCB):
                    sa_s[k] = jnp.where(new, vs[k], sa_s[k] + vs[k])
                    ma_s[k] = jnp.where(new, vs[k], jnp.maximum(ma_s[k], vs[k]))
                p = sid
            for k in range(NCB):
                sacc[pl.ds(k * L, L)] = sa_s[k]
                macc[pl.ds(k * L, L)] = ma_s[k]

        pseg[0] = idvec[L - 1]

    def process(xb, ib):
        def gbody(gi, carry):
            group(xb, ib, gi, 2 * gi)
            return carry
        lax.fori_loop(0, GPB, gbody, 0)

    xstart(0, xbuf0, ibuf0, xsem0, isem0)

    def pair_body(p, carry):
        g0 = 2 * p
        g1 = g0 + 1

        @pl.when(g1 < nblk)
        def _():
            xstart(g1, xbuf1, ibuf1, xsem1, isem1)

        xwait(g0, xbuf0, ibuf0, xsem0, isem0)
        process(xbuf0, ibuf0)

        @pl.when(g0 + 2 < nblk)
        def _():
            xstart(g0 + 2, xbuf0, ibuf0, xsem0, isem0)

        @pl.when(g1 < nblk)
        def _2():
            xwait(g1, xbuf1, ibuf1, xsem1, isem1)
            process(xbuf1, ibuf1)

        return carry

    npair = (nblk + 1) // 2
    lax.fori_loop(0, npair, pair_body, 0)

    flush(pseg[0],
          [sacc[pl.ds(k * L, L)] for k in range(NCB)],
          [macc[pl.ds(k * L, L)] for k in range(NCB)])
    n = nflush[0]
    for slot, fb, fsem in ((0, fbufA, fsem0), (1, fbufB, fsem1)):
        @pl.when(jnp.logical_or(
            n >= 2, jnp.logical_and(n >= 1, lax.rem(n - 1, 2) == slot)))
        def _(fb=fb, fsem=fsem):
            pltpu.make_async_copy(
                fb, scr_hbm.at[pl.ds(pl.multiple_of(sbase, 8), 256)],
                fsem).wait()

    plsc.subcore_barrier()

    seg0 = rs * 64
    for half in range(2):
        segh = pl.multiple_of(seg0 + half * 32, 8)
        for sh in range(8):
            src = scr_hbm.at[pl.ds(
                pl.multiple_of((cg * 8 + sh) * SLAB + segh * 256, 8), 8192)]
            pltpu.async_copy(src, mbufs[sh], msem)
        for sh in range(8):
            src = scr_hbm.at[pl.ds(
                pl.multiple_of((cg * 8 + sh) * SLAB + segh * 256, 8), 8192)]
            pltpu.make_async_copy(src, mbufs[sh], msem).wait()

        def mrow(r, carry):
            for k in range(NCB):
                vals = [mbufs[sh][pl.ds(r * 256 + k * L, L)]
                        for sh in range(8)]
                osum2d[r, pl.ds(k * L, L)] = _tree(lambda x, y: x + y, vals)
                vals = [mbufs[sh][pl.ds(r * 256 + 128 + k * L, L)]
                        for sh in range(8)]
                omax2d[r, pl.ds(k * L, L)] = _tree(jnp.maximum, vals)
            return carry

        lax.fori_loop(0, 32, mrow, 0)
        pltpu.sync_copy(osum2d,
                        out_hbm.at[pl.ds(segh, 32), pl.ds(ccol, 128)])
        pltpu.sync_copy(
            omax2d,
            out_hbm.at[pl.ds(segh, 32),
                       pl.ds(pl.multiple_of(D + cg * 128, 128), 128)])


@jax.jit
def _readout(x, ids):
    x3 = x.reshape(NROWS // 8, 8, D)
    mesh = plsc.VectorSubcoreMesh(core_axis_name="c", subcore_axis_name="s")
    fn = pl.kernel(
        _sc_body,
        out_type=(jax.ShapeDtypeStruct((NSEG, 2 * D), jnp.float32),
                  jax.ShapeDtypeStruct((32 * SLAB,), jnp.float32)),
        mesh=mesh,
        scratch_types=[
            pltpu.VMEM((BI, 8, 128), jnp.float32),
            pltpu.VMEM((BI, 8, 128), jnp.float32),
            pltpu.VMEM((B,), jnp.int32),
            pltpu.VMEM((B,), jnp.int32),
            pltpu.VMEM((128,), jnp.float32),
            pltpu.VMEM((128,), jnp.float32),
            pltpu.VMEM((256,), jnp.float32),
            pltpu.VMEM((256,), jnp.float32),
            pltpu.VMEM((8192,), jnp.float32),
            pltpu.VMEM((32, 128), jnp.float32),
            pltpu.VMEM((32, 128), jnp.float32),
            pltpu.VMEM((8192,), jnp.float32),
            pltpu.VMEM((8192,), jnp.float32),
            pltpu.VMEM((8192,), jnp.float32),
            pltpu.VMEM((8192,), jnp.float32),
            pltpu.VMEM((8192,), jnp.float32),
            pltpu.VMEM((8192,), jnp.float32),
            pltpu.VMEM((8192,), jnp.float32),
            pltpu.VMEM((8192,), jnp.float32),
            pltpu.SMEM((1,), jnp.int32),
            pltpu.SMEM((1,), jnp.int32),
            pltpu.SemaphoreType.DMA,
            pltpu.SemaphoreType.DMA,
            pltpu.SemaphoreType.DMA,
            pltpu.SemaphoreType.DMA,
            pltpu.SemaphoreType.DMA,
            pltpu.SemaphoreType.DMA,
            pltpu.SemaphoreType.DMA,
        ],
    )
    out, _ = fn(x3, ids)
    return out


def kernel(x, batch):
    return _readout(x, batch.astype(jnp.int32))


# speedup vs baseline: 1.4254x; 1.1130x over previous
import jax
import jax.numpy as jnp
from jax import lax
from jax.experimental import pallas as pl
from jax.experimental.pallas import tpu as pltpu
from jax.experimental.pallas import tpu_sc as plsc

NROWS = 100000
D = 512
NSEG = 512
L = 16
NCB = 8
BI = 20
B = BI * 8
GPB = B // L
SHARDI = 1600
NBLK7 = (NROWS // 8 - 7 * SHARDI) // BI
NBLKX = SHARDI // BI
SLAB = NSEG * 2 * 128


def _sc_body(x_hbm, ids_hbm, out_hbm, scr_hbm,
             xbuf0, xbuf1, ibuf0, ibuf1, sacc, macc, fbufA, fbufB, obuf,
             osum2d, omax2d,
             mbufs0, mbufs1, mbufs2, mbufs3, mbufs4, mbufs5, mbufs6, mbufs7,
             nflush, pseg,
             xsem0, xsem1, isem0, isem1, fsem0, fsem1, msem):
    c = lax.axis_index("c")
    s = lax.axis_index("s")
    cg = c * 2 + s // 8
    rs = s % 8
    mbufs = [mbufs0, mbufs1, mbufs2, mbufs3, mbufs4, mbufs5, mbufs6, mbufs7]

    ccol = pl.multiple_of(cg * 128, 128)
    ibase = rs * SHARDI
    nblk = jnp.where(rs == 7, NBLK7, NBLKX)
    sbase = pl.multiple_of((cg * 8 + rs) * SLAB, 8)

    zeros = jnp.zeros((L,), jnp.float32)
    ninf = jnp.full((L,), -jnp.inf, jnp.float32)

    for r in range(32):
        for k in range(NCB):
            obuf[pl.ds(r * 256 + k * L, L)] = zeros
            obuf[pl.ds(r * 256 + 128 + k * L, L)] = ninf
    for chunk in range(16):
        pltpu.sync_copy(
            obuf,
            scr_hbm.at[pl.ds(pl.multiple_of(sbase + chunk * 8192, 8), 8192)])
    nflush[0] = 0
    pseg[0] = -1
    for k in range(NCB):
        sacc[pl.ds(k * L, L)] = zeros
        macc[pl.ds(k * L, L)] = ninf

    def xstart(g, xb, ib, xsem, isem):
        i0 = ibase + g * BI
        pltpu.async_copy(x_hbm.at[pl.ds(i0, BI), :, pl.ds(ccol, 128)],
                         xb, xsem)
        pltpu.async_copy(
            ids_hbm.at[pl.ds(pl.multiple_of(i0 * 8, 8), B)], ib, isem)

    def xwait(g, xb, ib, xsem, isem):
        i0 = ibase + g * BI
        pltpu.make_async_copy(
            x_hbm.at[pl.ds(i0, BI), :, pl.ds(ccol, 128)], xb, xsem).wait()
        pltpu.make_async_copy(
            ids_hbm.at[pl.ds(pl.multiple_of(i0 * 8, 8), B)], ib, isem).wait()

    def _tree(op, xs):
        while len(xs) > 1:
            xs = [op(xs[i], xs[i + 1]) for i in range(0, len(xs), 2)]
        return xs[0]

    def flush(seg, svals, mvals):
        n = nflush[0]
        dst = scr_hbm.at[pl.ds(pl.multiple_of(sbase + seg * 256, 8), 256)]
        for slot, fb, fsem in ((0, fbufA, fsem0), (1, fbufB, fsem1)):
            @pl.when(lax.rem(n, 2) == slot)
            def _(fb=fb, fsem=fsem):
                @pl.when(n >= 2)
                def _():
                    pltpu.make_async_copy(fb, dst, fsem).wait()

                for k in range(NCB):
                    fb[pl.ds(k * L, L)] = svals[k]
                    fb[pl.ds(128 + k * L, L)] = mvals[k]
                pltpu.async_copy(fb, dst, fsem)
        nflush[0] = n + 1

    def group(xb, ib, gi, i2):
        idvec = ib[pl.ds(gi * L, L)]
        prev = pseg[0]
        same = jnp.logical_and(idvec[0] == idvec[L - 1], idvec[0] == prev)

        @pl.when(same)
        def fast():
            for k in range(0, NCB, 2):
                cols = []
                for kk in (k, k + 1):
                    cols.append([xb[i2 + a, r, pl.ds(kk * L, L)]
                                 for a in range(2) for r in range(8)])
                sums = [list(cs) for cs in cols]
                maxs = [list(cs) for cs in cols]
                while len(sums[0]) > 1:
                    sums = [[x + y for x, y in zip(cs[::2], cs[1::2])]
                            for cs in sums]
                    maxs = [[jnp.maximum(x, y) for x, y in zip(cs[::2], cs[1::2])]
                            for cs in maxs]
                for kk, si, mi in ((k, 0, 0), (k + 1, 1, 1)):
                    plsc.addupdate(sacc.at[pl.ds(kk * L, L)], sums[si][0])
                    macc[pl.ds(kk * L, L)] = jnp.maximum(
                        macc[pl.ds(kk * L, L)], maxs[mi][0])

        @pl.when(jnp.logical_not(same))
        def slow():
            p = prev
            sa_s = [sacc[pl.ds(k * L, L)] for k in range(NCB)]
            ma_s = [macc[pl.ds(k * L, L)] for k in range(NCB)]
            for j in range(L):
                a, r = j // 8, j % 8
                sid = idvec[j]
                new = sid != p

                @pl.when(jnp.logical_and(new, p >= 0))
                def _(sa_s=tuple(sa_s), ma_s=tuple(ma_s), p=p):
                    flush(p, sa_s, ma_s)

                vs = [xb[i2 + a, r, pl.ds(k * L, L)] for k in range(NCB)]
                for k in range(NCB):
                    sa_s[k] = jnp.where(new, vs[k], sa_s[k] + vs[k])
                    ma_s[k] = jnp.where(new, vs[k], jnp.maximum(ma_s[k], vs[k]))
                p = sid
            for k in range(NCB):
                sacc[pl.ds(k * L, L)] = sa_s[k]
                macc[pl.ds(k * L, L)] = ma_s[k]

        pseg[0] = idvec[L - 1]

    def process(xb, ib):
        def gbody(gi, carry):
            group(xb, ib, gi, 2 * gi)
            return carry
        lax.fori_loop(0, GPB, gbody, 0)

    xstart(0, xbuf0, ibuf0, xsem0, isem0)

    def pair_body(p, carry):
        g0 = 2 * p
        g1 = g0 + 1

        @pl.when(g1 < nblk)
        def _():
            xstart(g1, xbuf1, ibuf1, xsem1, isem1)

        xwait(g0, xbuf0, ibuf0, xsem0, isem0)
        process(xbuf0, ibuf0)

        @pl.when(g0 + 2 < nblk)
        def _():
            xstart(g0 + 2, xbuf0, ibuf0, xsem0, isem0)

        @pl.when(g1 < nblk)
        def _2():
            xwait(g1, xbuf1, ibuf1, xsem1, isem1)
            process(xbuf1, ibuf1)

        return carry

    npair = (nblk + 1) // 2
    lax.fori_loop(0, npair, pair_body, 0)

    flush(pseg[0],
          [sacc[pl.ds(k * L, L)] for k in range(NCB)],
          [macc[pl.ds(k * L, L)] for k in range(NCB)])
    n = nflush[0]
    for slot, fb, fsem in ((0, fbufA, fsem0), (1, fbufB, fsem1)):
        @pl.when(jnp.logical_or(
            n >= 2, jnp.logical_and(n >= 1, lax.rem(n - 1, 2) == slot)))
        def _(fb=fb, fsem=fsem):
            pltpu.make_async_copy(
                fb, scr_hbm.at[pl.ds(pl.multiple_of(sbase, 8), 256)],
                fsem).wait()

    plsc.subcore_barrier()

    seg0 = rs * 64
    for half in range(2):
        segh = pl.multiple_of(seg0 + half * 32, 8)
        for sh in range(8):
            src = scr_hbm.at[pl.ds(
                pl.multiple_of((cg * 8 + sh) * SLAB + segh * 256, 8), 8192)]
            pltpu.async_copy(src, mbufs[sh], msem)
        for sh in range(8):
            src = scr_hbm.at[pl.ds(
                pl.multiple_of((cg * 8 + sh) * SLAB + segh * 256, 8), 8192)]
            pltpu.make_async_copy(src, mbufs[sh], msem).wait()

        def mrow(r, carry):
            for k in range(NCB):
                vals = [mbufs[sh][pl.ds(r * 256 + k * L, L)]
                        for sh in range(8)]
                osum2d[r, pl.ds(k * L, L)] = _tree(lambda x, y: x + y, vals)
                vals = [mbufs[sh][pl.ds(r * 256 + 128 + k * L, L)]
                        for sh in range(8)]
                omax2d[r, pl.ds(k * L, L)] = _tree(jnp.maximum, vals)
            return carry

        lax.fori_loop(0, 32, mrow, 0)
        pltpu.sync_copy(osum2d,
                        out_hbm.at[pl.ds(segh, 32), pl.ds(ccol, 128)])
        pltpu.sync_copy(
            omax2d,
            out_hbm.at[pl.ds(segh, 32),
                       pl.ds(pl.multiple_of(D + cg * 128, 128), 128)])


@jax.jit
def _readout(x, ids):
    x3 = x.reshape(NROWS // 8, 8, D)
    mesh = plsc.VectorSubcoreMesh(core_axis_name="c", subcore_axis_name="s")
    fn = pl.kernel(
        _sc_body,
        out_type=(jax.ShapeDtypeStruct((NSEG, 2 * D), jnp.float32),
                  jax.ShapeDtypeStruct((32 * SLAB,), jnp.float32)),
        mesh=mesh,
        scratch_types=[
            pltpu.VMEM((BI, 8, 128), jnp.float32),
            pltpu.VMEM((BI, 8, 128), jnp.float32),
            pltpu.VMEM((B,), jnp.int32),
            pltpu.VMEM((B,), jnp.int32),
            pltpu.VMEM((128,), jnp.float32),
            pltpu.VMEM((128,), jnp.float32),
            pltpu.VMEM((256,), jnp.float32),
            pltpu.VMEM((256,), jnp.float32),
            pltpu.VMEM((8192,), jnp.float32),
            pltpu.VMEM((32, 128), jnp.float32),
            pltpu.VMEM((32, 128), jnp.float32),
            pltpu.VMEM((8192,), jnp.float32),
            pltpu.VMEM((8192,), jnp.float32),
            pltpu.VMEM((8192,), jnp.float32),
            pltpu.VMEM((8192,), jnp.float32),
            pltpu.VMEM((8192,), jnp.float32),
            pltpu.VMEM((8192,), jnp.float32),
            pltpu.VMEM((8192,), jnp.float32),
            pltpu.VMEM((8192,), jnp.float32),
            pltpu.SMEM((1,), jnp.int32),
            pltpu.SMEM((1,), jnp.int32),
            pltpu.SemaphoreType.DMA,
            pltpu.SemaphoreType.DMA,
            pltpu.SemaphoreType.DMA,
            pltpu.SemaphoreType.DMA,
            pltpu.SemaphoreType.DMA,
            pltpu.SemaphoreType.DMA,
            pltpu.SemaphoreType.DMA,
        ],
    )
    out, _ = fn(x3, ids)
    return out


def kernel(x, batch):
    return _readout(x, batch.astype(jnp.int32))


# speedup vs baseline: 1.4726x; 1.0331x over previous
import jax
import jax.numpy as jnp
from jax import lax
from jax.experimental import pallas as pl
from jax.experimental.pallas import tpu as pltpu
from jax.experimental.pallas import tpu_sc as plsc

NROWS = 100000
D = 512
NSEG = 512
L = 16
NCB = 8
BI = 20
B = BI * 8
GPB = B // L
SHARDI = 1600
NBLK7 = (NROWS // 8 - 7 * SHARDI) // BI
NBLKX = SHARDI // BI
SLAB = NSEG * 2 * 128


def _sc_body(x_hbm, ids_hbm, out_hbm, scr_hbm,
             xbuf0, xbuf1, ibuf0, ibuf1, sacc, macc, fbufA, fbufB, obuf,
             osum2d, omax2d,
             mbufs0, mbufs1, mbufs2, mbufs3, mbufs4, mbufs5, mbufs6, mbufs7,
             nflush, pseg,
             xsem0, xsem1, isem0, isem1, fsem0, fsem1, msem):
    c = lax.axis_index("c")
    s = lax.axis_index("s")
    cg = c * 2 + s // 8
    rs = s % 8
    mbufs = [mbufs0, mbufs1, mbufs2, mbufs3, mbufs4, mbufs5, mbufs6, mbufs7]

    ccol = pl.multiple_of(cg * 128, 128)
    ibase = rs * SHARDI
    nblk = jnp.where(rs == 7, NBLK7, NBLKX)
    sbase = pl.multiple_of((cg * 8 + rs) * SLAB, 8)

    zeros = jnp.zeros((L,), jnp.float32)
    ninf = jnp.full((L,), -jnp.inf, jnp.float32)

    for r in range(32):
        for k in range(NCB):
            obuf[pl.ds(r * 256 + k * L, L)] = zeros
            obuf[pl.ds(r * 256 + 128 + k * L, L)] = ninf
    for chunk in range(16):
        pltpu.sync_copy(
            obuf,
            scr_hbm.at[pl.ds(pl.multiple_of(sbase + chunk * 8192, 8), 8192)])
    nflush[0] = 0
    pseg[0] = -1
    for k in range(NCB):
        sacc[pl.ds(k * L, L)] = zeros
        macc[pl.ds(k * L, L)] = ninf

    def xstart(g, xb, ib, xsem, isem):
        i0 = ibase + g * BI
        pltpu.async_copy(x_hbm.at[pl.ds(i0, BI), :, pl.ds(ccol, 128)],
                         xb, xsem)
        pltpu.async_copy(
            ids_hbm.at[pl.ds(pl.multiple_of(i0 * 8, 8), B)], ib, isem)

    def xwait(g, xb, ib, xsem, isem):
        i0 = ibase + g * BI
        pltpu.make_async_copy(
            x_hbm.at[pl.ds(i0, BI), :, pl.ds(ccol, 128)], xb, xsem).wait()
        pltpu.make_async_copy(
            ids_hbm.at[pl.ds(pl.multiple_of(i0 * 8, 8), B)], ib, isem).wait()

    def _tree(op, xs):
        while len(xs) > 1:
            xs = [op(xs[i], xs[i + 1]) for i in range(0, len(xs), 2)]
        return xs[0]

    def flush(seg, svals, mvals):
        n = nflush[0]
        dst = scr_hbm.at[pl.ds(pl.multiple_of(sbase + seg * 256, 8), 256)]
        for slot, fb, fsem in ((0, fbufA, fsem0), (1, fbufB, fsem1)):
            @pl.when(lax.rem(n, 2) == slot)
            def _(fb=fb, fsem=fsem):
                @pl.when(n >= 2)
                def _():
                    pltpu.make_async_copy(fb, dst, fsem).wait()

                for k in range(NCB):
                    fb[pl.ds(k * L, L)] = svals[k]
                    fb[pl.ds(128 + k * L, L)] = mvals[k]
                pltpu.async_copy(fb, dst, fsem)
        nflush[0] = n + 1

    def group(xb, ib, gi, i2):
        idvec = ib[pl.ds(gi * L, L)]
        prev = pseg[0]
        same = jnp.logical_and(idvec[0] == idvec[L - 1], idvec[0] == prev)

        @pl.when(same)
        def fast():
            for k in range(0, NCB, 4):
                cols = []
                for kk in range(k, k + 4):
                    cols.append([xb[i2 + a, r, pl.ds(kk * L, L)]
                                 for a in range(2) for r in range(8)])
                sums = [list(cs) for cs in cols]
                maxs = [list(cs) for cs in cols]
                while len(sums[0]) > 1:
                    sums = [[x + y for x, y in zip(cs[::2], cs[1::2])]
                            for cs in sums]
                    maxs = [[jnp.maximum(x, y) for x, y in zip(cs[::2], cs[1::2])]
                            for cs in maxs]
                for i, kk in enumerate(range(k, k + 4)):
                    plsc.addupdate(sacc.at[pl.ds(kk * L, L)], sums[i][0])
                    macc[pl.ds(kk * L, L)] = jnp.maximum(
                        macc[pl.ds(kk * L, L)], maxs[i][0])

        @pl.when(jnp.logical_not(same))
        def slow():
            p = prev
            sa_s = [sacc[pl.ds(k * L, L)] for k in range(NCB)]
            ma_s = [macc[pl.ds(k * L, L)] for k in range(NCB)]
            for j in range(L):
                a, r = j // 8, j % 8
                sid = idvec[j]
                new = sid != p

                @pl.when(jnp.logical_and(new, p >= 0))
                def _(sa_s=tuple(sa_s), ma_s=tuple(ma_s), p=p):
                    flush(p, sa_s, ma_s)

                vs = [xb[i2 + a, r, pl.ds(k * L, L)] for k in range(NCB)]
                for k in range(NCB):
                    sa_s[k] = jnp.where(new, vs[k], sa_s[k] + vs[k])
                    ma_s[k] = jnp.where(new, vs[k], jnp.maximum(ma_s[k], vs[k]))
                p = sid
            for k in range(NCB):
                sacc[pl.ds(k * L, L)] = sa_s[k]
                macc[pl.ds(k * L, L)] = ma_s[k]

        pseg[0] = idvec[L - 1]

    def process(xb, ib):
        def gbody(gi, carry):
            group(xb, ib, gi, 2 * gi)
            return carry
        lax.fori_loop(0, GPB, gbody, 0)

    xstart(0, xbuf0, ibuf0, xsem0, isem0)

    def pair_body(p, carry):
        g0 = 2 * p
        g1 = g0 + 1

        @pl.when(g1 < nblk)
        def _():
            xstart(g1, xbuf1, ibuf1, xsem1, isem1)

        xwait(g0, xbuf0, ibuf0, xsem0, isem0)
        process(xbuf0, ibuf0)

        @pl.when(g0 + 2 < nblk)
        def _():
            xstart(g0 + 2, xbuf0, ibuf0, xsem0, isem0)

        @pl.when(g1 < nblk)
        def _2():
            xwait(g1, xbuf1, ibuf1, xsem1, isem1)
            process(xbuf1, ibuf1)

        return carry

    npair = (nblk + 1) // 2
    lax.fori_loop(0, npair, pair_body, 0)

    flush(pseg[0],
          [sacc[pl.ds(k * L, L)] for k in range(NCB)],
          [macc[pl.ds(k * L, L)] for k in range(NCB)])
    n = nflush[0]
    for slot, fb, fsem in ((0, fbufA, fsem0), (1, fbufB, fsem1)):
        @pl.when(jnp.logical_or(
            n >= 2, jnp.logical_and(n >= 1, lax.rem(n - 1, 2) == slot)))
        def _(fb=fb, fsem=fsem):
            pltpu.make_async_copy(
                fb, scr_hbm.at[pl.ds(pl.multiple_of(sbase, 8), 256)],
                fsem).wait()

    plsc.subcore_barrier()

    seg0 = rs * 64
    for half in range(2):
        segh = pl.multiple_of(seg0 + half * 32, 8)
        for sh in range(8):
            src = scr_hbm.at[pl.ds(
                pl.multiple_of((cg * 8 + sh) * SLAB + segh * 256, 8), 8192)]
            pltpu.async_copy(src, mbufs[sh], msem)
        for sh in range(8):
            src = scr_hbm.at[pl.ds(
                pl.multiple_of((cg * 8 + sh) * SLAB + segh * 256, 8), 8192)]
            pltpu.make_async_copy(src, mbufs[sh], msem).wait()

        def mrow(r, carry):
            for k in range(NCB):
                vals = [mbufs[sh][pl.ds(r * 256 + k * L, L)]
                        for sh in range(8)]
                osum2d[r, pl.ds(k * L, L)] = _tree(lambda x, y: x + y, vals)
                vals = [mbufs[sh][pl.ds(r * 256 + 128 + k * L, L)]
                        for sh in range(8)]
                omax2d[r, pl.ds(k * L, L)] = _tree(jnp.maximum, vals)
            return carry

        lax.fori_loop(0, 32, mrow, 0)
        pltpu.sync_copy(osum2d,
                        out_hbm.at[pl.ds(segh, 32), pl.ds(ccol, 128)])
        pltpu.sync_copy(
            omax2d,
            out_hbm.at[pl.ds(segh, 32),
                       pl.ds(pl.multiple_of(D + cg * 128, 128), 128)])


@jax.jit
def _readout(x, ids):
    x3 = x.reshape(NROWS // 8, 8, D)
    mesh = plsc.VectorSubcoreMesh(core_axis_name="c", subcore_axis_name="s")
    fn = pl.kernel(
        _sc_body,
        out_type=(jax.ShapeDtypeStruct((NSEG, 2 * D), jnp.float32),
                  jax.ShapeDtypeStruct((32 * SLAB,), jnp.float32)),
        mesh=mesh,
        scratch_types=[
            pltpu.VMEM((BI, 8, 128), jnp.float32),
            pltpu.VMEM((BI, 8, 128), jnp.float32),
            pltpu.VMEM((B,), jnp.int32),
            pltpu.VMEM((B,), jnp.int32),
            pltpu.VMEM((128,), jnp.float32),
            pltpu.VMEM((128,), jnp.float32),
            pltpu.VMEM((256,), jnp.float32),
            pltpu.VMEM((256,), jnp.float32),
            pltpu.VMEM((8192,), jnp.float32),
            pltpu.VMEM((32, 128), jnp.float32),
            pltpu.VMEM((32, 128), jnp.float32),
            pltpu.VMEM((8192,), jnp.float32),
            pltpu.VMEM((8192,), jnp.float32),
            pltpu.VMEM((8192,), jnp.float32),
            pltpu.VMEM((8192,), jnp.float32),
            pltpu.VMEM((8192,), jnp.float32),
            pltpu.VMEM((8192,), jnp.float32),
            pltpu.VMEM((8192,), jnp.float32),
            pltpu.VMEM((8192,), jnp.float32),
            pltpu.SMEM((1,), jnp.int32),
            pltpu.SMEM((1,), jnp.int32),
            pltpu.SemaphoreType.DMA,
            pltpu.SemaphoreType.DMA,
            pltpu.SemaphoreType.DMA,
            pltpu.SemaphoreType.DMA,
            pltpu.SemaphoreType.DMA,
            pltpu.SemaphoreType.DMA,
            pltpu.SemaphoreType.DMA,
        ],
    )
    out, _ = fn(x3, ids)
    return out


def kernel(x, batch):
    return _readout(x, batch.astype(jnp.int32))
